# baseline (speedup 1.0000x reference)
import functools

import jax
import jax.numpy as jnp
from jax import lax
from jax.experimental import pallas as pl
from jax.experimental.pallas import tpu as pltpu
from jax.experimental.pallas import tpu_sc as plsc

_N = 10000
_E = 320000
_D = 128

_NC = 2
_NS = 16
_NW = _NC * _NS
_C = 96
_NCHUNK = 105
_EPT = _NCHUNK * _C
_EPAD = _NW * _EPT
_NPAD = 8
_NA = _N + _NPAD
_RPT = 632
_RPT_LAST = _N - (_NS - 1) * _RPT
_ZPT_LAST = _NA - (_NS - 1) * _RPT


@functools.lru_cache(maxsize=None)
def _make_seg_sum(with_counts: bool):
    mesh = plsc.VectorSubcoreMesh(core_axis_name="c", subcore_axis_name="s",
                                  num_cores=_NC, num_subcores=_NS)
    out_type = [jax.ShapeDtypeStruct((_NC, _N, _D), jnp.float32)]
    scratch = [
        pltpu.VMEM_SHARED((_NA, _D), jnp.float32),
        pltpu.VMEM((2, _C), jnp.int32),
        pltpu.VMEM((_NCHUNK, _C), jnp.int32),
        pltpu.VMEM((2, _C, _D), jnp.float32),
        pltpu.SemaphoreType.DMA,
        pltpu.SemaphoreType.DMA,
        pltpu.SemaphoreType.DMA,
    ]
    if with_counts:
        out_type.append(jax.ShapeDtypeStruct((_NC, _NA), jnp.float32))
        scratch.append(pltpu.VMEM_SHARED((_NA,), jnp.float32))
        scratch.append(pltpu.VMEM((128,), jnp.float32))
        scratch.append(pltpu.SemaphoreType.DMA)

    @functools.partial(
        pl.kernel,
        out_type=tuple(out_type),
        mesh=mesh,
        scratch_types=tuple(scratch),
    )
    def seg_sum(x_hbm, src_hbm, dst_hbm, zrows_hbm, zn_hbm, *rest):
        if with_counts:
            (sum_out, cnt_out, acc, sidx, dst_v, rows_v, isem, gsem, ssem,
             cnt, ones_v, csem) = rest
        else:
            sum_out, acc, sidx, dst_v, rows_v, isem, gsem, ssem = rest
        c = lax.axis_index("c")
        s = lax.axis_index("s")
        w = s * _NC + c
        ebase = w * _EPT

        base = pl.multiple_of(s * _RPT, 8)

        @pl.when(s < _NS - 1)
        def _():
            pltpu.sync_copy(zrows_hbm.at[pl.ds(0, _RPT)], acc.at[pl.ds(base, _RPT)])

        @pl.when(s == _NS - 1)
        def _():
            pltpu.sync_copy(zrows_hbm.at[pl.ds(0, _ZPT_LAST)],
                            acc.at[pl.ds(base, _ZPT_LAST)])
        if with_counts:

            @pl.when(s == 0)
            def _():
                pltpu.sync_copy(zn_hbm, cnt)

            for k in range(8):
                ones_v[pl.ds(k * 16, 16)] = jnp.ones((16,), jnp.float32)

        pltpu.sync_copy(dst_hbm.at[w], dst_v)
        plsc.subcore_barrier()

        def src_chunk(j):
            return src_hbm.at[pl.ds(pl.multiple_of(ebase + j * _C, 8), _C)]

        pltpu.sync_copy(src_chunk(0), sidx.at[0])
        pltpu.async_copy(x_hbm.at[sidx.at[0]], rows_v.at[0], gsem)
        pltpu.async_copy(src_chunk(1), sidx.at[1], isem)

        def chunk_body(j, carry):
            m = lax.rem(j, 2)

            @pl.when(j > 0)
            def _():
                pltpu.make_async_copy(rows_v.at[1 - m],
                                      acc.at[dst_v.at[j - 1]], ssem).wait()

            pltpu.make_async_copy(x_hbm.at[sidx.at[m]], rows_v.at[m], gsem).wait()
            pltpu.async_copy(rows_v.at[m], acc.at[dst_v.at[j]], ssem, add=True)
            if with_counts:
                pltpu.async_copy(ones_v.at[pl.ds(0, _C)], cnt.at[dst_v.at[j]],
                                 csem, add=True)

            @pl.when(j < _NCHUNK - 1)
            def _():
                pltpu.make_async_copy(src_chunk(j + 1), sidx.at[1 - m], isem).wait()
                pltpu.async_copy(x_hbm.at[sidx.at[1 - m]], rows_v.at[1 - m], gsem)

            @pl.when(j < _NCHUNK - 2)
            def _():
                pltpu.async_copy(src_chunk(j + 2), sidx.at[m], isem)

            return carry

        lax.fori_loop(0, _NCHUNK, chunk_body, 0)
        pltpu.make_async_copy(rows_v.at[(_NCHUNK - 1) % 2],
                              acc.at[dst_v.at[_NCHUNK - 1]], ssem).wait()
        if with_counts:
            def drain_body(j, carry):
                pltpu.make_async_copy(ones_v.at[pl.ds(0, _C)],
                                      cnt.at[dst_v.at[j]], csem).wait()
                return carry

            lax.fori_loop(0, _NCHUNK, drain_body, 0)
        plsc.subcore_barrier()

        @pl.when(s < _NS - 1)
        def _():
            pltpu.sync_copy(acc.at[pl.ds(base, _RPT)],
                            sum_out.at[c, pl.ds(base, _RPT)])

        @pl.when(s == _NS - 1)
        def _():
            pltpu.sync_copy(acc.at[pl.ds(base, _RPT_LAST)],
                            sum_out.at[c, pl.ds(base, _RPT_LAST)])
        if with_counts:

            @pl.when(s == 0)
            def _():
                pltpu.sync_copy(cnt, cnt_out.at[c])

    return seg_sum


def _dense1_body(parts, cnts, x, w_l_t, b_l, w_r_t, gamma, beta, h_ref, cinv_ref):
    cnt = cnts[0] + cnts[1]
    cinv = 1.0 / jnp.maximum(cnt, 1.0)
    agg = (parts[0] + parts[1]) * cinv
    h = (jnp.dot(agg, w_l_t[...], preferred_element_type=jnp.float32)
         + b_l[...]
         + jnp.dot(x[...], w_r_t[...], preferred_element_type=jnp.float32))
    m = jnp.mean(h, axis=0, keepdims=True)
    v = jnp.mean((h - m) * (h - m), axis=0, keepdims=True)
    h = (h - m) * lax.rsqrt(v + 1e-5) * gamma[...] + beta[...]
    h_ref[...] = jnp.where(h >= 0, h, 0.01 * h)
    cinv_ref[...] = cinv


def _dense2_body(parts, cinv, h, w_l_t, b_l, w_r_t, out_ref):
    agg = (parts[0] + parts[1]) * cinv[...]
    out_ref[...] = (jnp.dot(agg, w_l_t[...], preferred_element_type=jnp.float32)
                    + b_l[...]
                    + jnp.dot(h[...], w_r_t[...], preferred_element_type=jnp.float32))


_dense1 = pl.pallas_call(
    _dense1_body,
    out_shape=(jax.ShapeDtypeStruct((_N, _D), jnp.float32),
               jax.ShapeDtypeStruct((_N, 1), jnp.float32)),
)

_dense2 = pl.pallas_call(
    _dense2_body,
    out_shape=jax.ShapeDtypeStruct((_N, _D), jnp.float32),
)


def kernel(x, edge_index, W1_l, b1_l, W1_r, bn_gamma, bn_beta, W2_l, b2_l, W2_r):
    pad = _EPAD - _E
    src = jnp.concatenate([edge_index[0], jnp.zeros((pad,), jnp.int32)])
    dst = jnp.concatenate(
        [edge_index[1], jnp.full((pad,), _N, jnp.int32)]
    ).reshape(_NW, _NCHUNK, _C)
    zrows = jnp.zeros((_RPT, _D), jnp.float32)
    zn = jnp.zeros((_NA,), jnp.float32)

    parts1, cnts = _make_seg_sum(True)(x, src, dst, zrows, zn)
    h, cinv = _dense1(parts1, cnts[:, :_N, None], x, W1_l.T, b1_l[None, :],
                      W1_r.T, bn_gamma[None, :], bn_beta[None, :])
    (parts2,) = _make_seg_sum(False)(h, src, dst, zrows, zn)
    out = _dense2(parts2, cinv, h, W2_l.T, b2_l[None, :], W2_r.T)
    return out


# speedup vs baseline: 1.0284x; 1.0284x over previous
import functools

import jax
import jax.numpy as jnp
from jax import lax
from jax.experimental import pallas as pl
from jax.experimental.pallas import tpu as pltpu
from jax.experimental.pallas import tpu_sc as plsc

_N = 10000
_E = 320000
_D = 128

_NC = 2
_NS = 16
_NW = _NC * _NS
_C = 120
_NCHUNK = 84
_EPT = _NCHUNK * _C
_EPAD = _NW * _EPT
_NPAD = 8
_NA = _N + _NPAD
_RPT = 632
_RPT_LAST = _N - (_NS - 1) * _RPT
_ZPT_LAST = _NA - (_NS - 1) * _RPT


@functools.lru_cache(maxsize=None)
def _make_seg_sum(with_counts: bool):
    mesh = plsc.VectorSubcoreMesh(core_axis_name="c", subcore_axis_name="s",
                                  num_cores=_NC, num_subcores=_NS)
    out_type = [jax.ShapeDtypeStruct((_NC, _N, _D), jnp.float32)]
    scratch = [
        pltpu.VMEM_SHARED((_NA, _D), jnp.float32),
        pltpu.VMEM((2, _C), jnp.int32),
        pltpu.VMEM((_NCHUNK, _C), jnp.int32),
        pltpu.VMEM((2, _C, _D), jnp.float32),
        pltpu.SemaphoreType.DMA,
        pltpu.SemaphoreType.DMA,
        pltpu.SemaphoreType.DMA,
    ]
    if with_counts:
        out_type.append(jax.ShapeDtypeStruct((_NC, _NA), jnp.float32))
        scratch.append(pltpu.VMEM_SHARED((_NA,), jnp.float32))
        scratch.append(pltpu.VMEM((128,), jnp.float32))
        scratch.append(pltpu.SemaphoreType.DMA)

    @functools.partial(
        pl.kernel,
        out_type=tuple(out_type),
        mesh=mesh,
        scratch_types=tuple(scratch),
    )
    def seg_sum(x_hbm, src_hbm, dst_hbm, zrows_hbm, zn_hbm, *rest):
        if with_counts:
            (sum_out, cnt_out, acc, sidx, dst_v, rows_v, isem, gsem, ssem,
             cnt, ones_v, csem) = rest
        else:
            sum_out, acc, sidx, dst_v, rows_v, isem, gsem, ssem = rest
        c = lax.axis_index("c")
        s = lax.axis_index("s")
        w = s * _NC + c
        ebase = w * _EPT

        base = pl.multiple_of(s * _RPT, 8)

        @pl.when(s < _NS - 1)
        def _():
            pltpu.sync_copy(zrows_hbm.at[pl.ds(0, _RPT)], acc.at[pl.ds(base, _RPT)])

        @pl.when(s == _NS - 1)
        def _():
            pltpu.sync_copy(zrows_hbm.at[pl.ds(0, _ZPT_LAST)],
                            acc.at[pl.ds(base, _ZPT_LAST)])
        if with_counts:

            @pl.when(s == 0)
            def _():
                pltpu.sync_copy(zn_hbm, cnt)

            for k in range(8):
                ones_v[pl.ds(k * 16, 16)] = jnp.ones((16,), jnp.float32)

        pltpu.sync_copy(dst_hbm.at[w], dst_v)
        plsc.subcore_barrier()

        def src_chunk(j):
            return src_hbm.at[pl.ds(pl.multiple_of(ebase + j * _C, 8), _C)]

        pltpu.sync_copy(src_chunk(0), sidx.at[0])
        pltpu.async_copy(src_chunk(1), sidx.at[1], isem)

        def chunk_body(j, carry):
            m = lax.rem(j, 2)
            pltpu.async_copy(x_hbm.at[sidx.at[m]], rows_v.at[m], gsem)

            @pl.when(j > 0)
            def _():
                pltpu.make_async_copy(rows_v.at[1 - m],
                                      acc.at[dst_v.at[j - 1]], ssem).wait()

            pltpu.make_async_copy(x_hbm.at[sidx.at[m]], rows_v.at[m], gsem).wait()
            pltpu.async_copy(rows_v.at[m], acc.at[dst_v.at[j]], ssem, add=True)
            if with_counts:
                pltpu.async_copy(ones_v.at[pl.ds(0, _C)], cnt.at[dst_v.at[j]],
                                 csem, add=True)

            @pl.when(j < _NCHUNK - 1)
            def _():
                pltpu.make_async_copy(src_chunk(j + 1), sidx.at[1 - m], isem).wait()

            @pl.when(j < _NCHUNK - 2)
            def _():
                pltpu.async_copy(src_chunk(j + 2), sidx.at[m], isem)

            return carry

        lax.fori_loop(0, _NCHUNK, chunk_body, 0)
        pltpu.make_async_copy(rows_v.at[(_NCHUNK - 1) % 2],
                              acc.at[dst_v.at[_NCHUNK - 1]], ssem).wait()
        if with_counts:
            def drain_body(j, carry):
                pltpu.make_async_copy(ones_v.at[pl.ds(0, _C)],
                                      cnt.at[dst_v.at[j]], csem).wait()
                return carry

            lax.fori_loop(0, _NCHUNK, drain_body, 0)
        plsc.subcore_barrier()

        @pl.when(s < _NS - 1)
        def _():
            pltpu.sync_copy(acc.at[pl.ds(base, _RPT)],
                            sum_out.at[c, pl.ds(base, _RPT)])

        @pl.when(s == _NS - 1)
        def _():
            pltpu.sync_copy(acc.at[pl.ds(base, _RPT_LAST)],
                            sum_out.at[c, pl.ds(base, _RPT_LAST)])
        if with_counts:

            @pl.when(s == 0)
            def _():
                pltpu.sync_copy(cnt, cnt_out.at[c])

    return seg_sum


def _dense1_body(parts, cnts, x, w_l_t, b_l, w_r_t, gamma, beta, h_ref, cinv_ref):
    cnt = cnts[0] + cnts[1]
    cinv = 1.0 / jnp.maximum(cnt, 1.0)
    agg = (parts[0] + parts[1]) * cinv
    h = (jnp.dot(agg, w_l_t[...], preferred_element_type=jnp.float32)
         + b_l[...]
         + jnp.dot(x[...], w_r_t[...], preferred_element_type=jnp.float32))
    m = jnp.mean(h, axis=0, keepdims=True)
    v = jnp.mean((h - m) * (h - m), axis=0, keepdims=True)
    h = (h - m) * lax.rsqrt(v + 1e-5) * gamma[...] + beta[...]
    h_ref[...] = jnp.where(h >= 0, h, 0.01 * h)
    cinv_ref[...] = cinv


def _dense2_body(parts, cinv, h, w_l_t, b_l, w_r_t, out_ref):
    agg = (parts[0] + parts[1]) * cinv[...]
    out_ref[...] = (jnp.dot(agg, w_l_t[...], preferred_element_type=jnp.float32)
                    + b_l[...]
                    + jnp.dot(h[...], w_r_t[...], preferred_element_type=jnp.float32))


_dense1 = pl.pallas_call(
    _dense1_body,
    out_shape=(jax.ShapeDtypeStruct((_N, _D), jnp.float32),
               jax.ShapeDtypeStruct((_N, 1), jnp.float32)),
)

_dense2 = pl.pallas_call(
    _dense2_body,
    out_shape=jax.ShapeDtypeStruct((_N, _D), jnp.float32),
)


def kernel(x, edge_index, W1_l, b1_l, W1_r, bn_gamma, bn_beta, W2_l, b2_l, W2_r):
    pad = _EPAD - _E
    src = jnp.concatenate([edge_index[0], jnp.zeros((pad,), jnp.int32)])
    dst = jnp.concatenate(
        [edge_index[1], jnp.full((pad,), _N, jnp.int32)]
    ).reshape(_NW, _NCHUNK, _C)
    zrows = jnp.zeros((_RPT, _D), jnp.float32)
    zn = jnp.zeros((_NA,), jnp.float32)

    parts1, cnts = _make_seg_sum(True)(x, src, dst, zrows, zn)
    h, cinv = _dense1(parts1, cnts[:, :_N, None], x, W1_l.T, b1_l[None, :],
                      W1_r.T, bn_gamma[None, :], bn_beta[None, :])
    (parts2,) = _make_seg_sum(False)(h, src, dst, zrows, zn)
    out = _dense2(parts2, cinv, h, W2_l.T, b2_l[None, :], W2_r.T)
    return out
